# baseline (speedup 1.0000x reference)
import functools

import jax
import jax.numpy as jnp
from jax import lax
from jax.experimental import pallas as pl
from jax.experimental.pallas import tpu as pltpu
from jax.experimental.pallas import tpu_sc as plsc

NC = 2
NS = 16
NW = NC * NS

HP = 64
G = 4
K = 2
R = 4


def _gather_body(rows_per_w, n_groups, hist, dim, idx_hbm, table_hbm, out_hbm,
                 idx_v, buf, g0, g1, g2, g3, o0, o1, o2, o3):
    gsem = [g0, g1, g2, g3]
    osem = [o0, o1, o2, o3]
    c = lax.axis_index("c")
    s = lax.axis_index("s")
    wid = s * NC + c
    base = wid * rows_per_w

    pltpu.sync_copy(idx_hbm.at[wid], idx_v)

    def fire_gather(g, slot):
        for ch in range(K):
            pltpu.async_copy(
                table_hbm.at[idx_v.at[g * K + ch]],
                buf.at[slot, pl.ds(ch * 2 * HP, 2 * HP)],
                gsem[slot],
            )

    def wait_gather(slot):
        pltpu.make_async_copy(
            table_hbm.at[pl.ds(0, G * HP)], buf.at[slot], gsem[slot]
        ).wait()

    def fire_out(g, slot):
        for r in range(G):
            pltpu.async_copy(
                buf.at[slot, pl.ds(r * HP, hist)],
                out_hbm.at[base + g * G + r],
                osem[slot],
            )

    def wait_out(slot):
        for r in range(G):
            pltpu.make_async_copy(
                buf.at[slot, pl.ds(0, hist)], out_hbm.at[0], osem[slot]
            ).wait()

    fire_gather(0, 0)
    fire_gather(1, 1)

    p_iters = n_groups // R

    def block(t, carry):
        for j in range(R):
            g = t * R + j
            s2 = (j + 2) % R
            if j < 2:
                @pl.when(t > 0)
                def _():
                    wait_out(s2)
                fire_gather(g + 2, s2)
            else:
                wait_out(s2)

                @pl.when(t < p_iters - 1)
                def _():
                    fire_gather(g + 2, s2)
            wait_gather(j)
            fire_out(g, j)
        return carry

    lax.fori_loop(0, p_iters, block, 0)
    wait_out(2)
    wait_out(3)


@jax.jit
def kernel(inputs, table):
    batch, hist = inputs.shape
    num_emb, dim = table.shape
    assert batch % (NW * G * R) == 0 and hist <= HP
    rows_per_w = batch // NW
    n_groups = rows_per_w // G
    n_chunks = rows_per_w // 2

    idx = jnp.pad(inputs.astype(jnp.int32), ((0, 0), (0, HP - hist)))
    idx = idx.reshape(NW, n_chunks, 2 * HP)

    mesh = plsc.VectorSubcoreMesh(core_axis_name="c", subcore_axis_name="s")
    out = pl.kernel(
        functools.partial(_gather_body, rows_per_w, n_groups, hist, dim),
        out_type=jax.ShapeDtypeStruct((batch, hist, dim), jnp.float32),
        mesh=mesh,
        scratch_types=[
            pltpu.VMEM((n_chunks, 2 * HP), jnp.int32),
            pltpu.VMEM((R, G * HP, dim), jnp.float32),
        ] + [pltpu.SemaphoreType.DMA] * (2 * R),
        compiler_params=pltpu.CompilerParams(use_tc_tiling_on_sc=False),
    )(idx, table)

    return out


# speedup vs baseline: 4.6352x; 4.6352x over previous
import functools

import jax
import jax.numpy as jnp
from jax import lax
from jax.experimental import pallas as pl
from jax.experimental.pallas import tpu as pltpu
from jax.experimental.pallas import tpu_sc as plsc

NC = 2
NS = 16
NW = NC * NS

HP = 64
G = 4
K = 2
R = 4


def _gather_body(rows_per_w, n_groups, hist, dim, idx_hbm, table_hbm, out_hbm,
                 idx_v, buf, g0, g1, g2, g3, o0, o1, o2, o3):
    gsem = [g0, g1, g2, g3]
    osem = [o0, o1, o2, o3]
    c = lax.axis_index("c")
    s = lax.axis_index("s")
    wid = s * NC + c
    base = wid * rows_per_w

    pltpu.sync_copy(idx_hbm.at[wid], idx_v)

    def fire_gather(g, slot):
        for ch in range(K):
            pltpu.async_copy(
                table_hbm.at[idx_v.at[g * K + ch]],
                buf.at[slot, pl.ds(ch * 2 * HP, 2 * HP)],
                gsem[slot],
            )

    def wait_gather(slot):
        pltpu.make_async_copy(
            table_hbm.at[pl.ds(0, G * HP)], buf.at[slot], gsem[slot]
        ).wait()

    def fire_out(g, slot):
        for r in range(G):
            pltpu.async_copy(
                buf.at[slot, pl.ds(r * HP, hist)],
                out_hbm.at[base + g * G + r],
                osem[slot],
            )

    def wait_out(slot):
        for r in range(G):
            pltpu.make_async_copy(
                buf.at[slot, pl.ds(0, hist)], out_hbm.at[0], osem[slot]
            ).wait()

    fire_gather(0, 0)
    fire_gather(1, 1)

    p_iters = n_groups // R

    def block(t, carry):
        for j in range(R):
            g = t * R + j
            s2 = (j + 2) % R
            if j < 2:
                @pl.when(t > 0)
                def _():
                    wait_out(s2)
                fire_gather(g + 2, s2)
            else:
                wait_out(s2)

                @pl.when(t < p_iters - 1)
                def _():
                    fire_gather(g + 2, s2)
            wait_gather(j)
            fire_out(g, j)
        return carry

    lax.fori_loop(0, p_iters, block, 0)
    wait_out(2)
    wait_out(3)


@jax.jit
def kernel(inputs, table):
    batch, hist = inputs.shape
    num_emb, dim = table.shape
    assert batch % (NW * G * R) == 0 and hist <= HP
    rows_per_w = batch // NW
    n_groups = rows_per_w // G
    n_chunks = rows_per_w // 2

    pad = (jnp.arange(batch, dtype=jnp.int32)[:, None] * 977
           + jnp.arange(HP - hist, dtype=jnp.int32)[None, :] * 131) % num_emb
    idx = jnp.concatenate([inputs.astype(jnp.int32), pad], axis=1)
    idx = idx.reshape(NW, n_chunks, 2 * HP)

    mesh = plsc.VectorSubcoreMesh(core_axis_name="c", subcore_axis_name="s")
    out = pl.kernel(
        functools.partial(_gather_body, rows_per_w, n_groups, hist, dim),
        out_type=jax.ShapeDtypeStruct((batch, hist, dim), jnp.float32),
        mesh=mesh,
        scratch_types=[
            pltpu.VMEM((n_chunks, 2 * HP), jnp.int32),
            pltpu.VMEM((R, G * HP, dim), jnp.float32),
        ] + [pltpu.SemaphoreType.DMA] * (2 * R),
        compiler_params=pltpu.CompilerParams(use_tc_tiling_on_sc=False),
    )(idx, table)

    return out


# speedup vs baseline: 4.7094x; 1.0160x over previous
import functools

import jax
import jax.numpy as jnp
from jax import lax
from jax.experimental import pallas as pl
from jax.experimental.pallas import tpu as pltpu
from jax.experimental.pallas import tpu_sc as plsc

NC = 2
NS = 16
NW = NC * NS

CHUNK = 128
K = 2
GROUP = K * CHUNK
R = 4


def _gather_body(n_groups, dim, idx_hbm, table_hbm, out_hbm,
                 idx_v, buf, g0, g1, g2, g3, o0, o1, o2, o3):
    gsem = [g0, g1, g2, g3]
    osem = [o0, o1, o2, o3]
    c = lax.axis_index("c")
    s = lax.axis_index("s")
    wid = s * NC + c

    pltpu.sync_copy(idx_hbm.at[wid], idx_v)

    def fire_gather(g, slot):
        for ch in range(K):
            pltpu.async_copy(
                table_hbm.at[idx_v.at[g * K + ch]],
                buf.at[slot, pl.ds(ch * CHUNK, CHUNK)],
                gsem[slot],
            )

    def wait_gather(slot):
        pltpu.make_async_copy(
            table_hbm.at[pl.ds(0, GROUP)], buf.at[slot], gsem[slot]
        ).wait()

    def fire_out(g, slot):
        pltpu.async_copy(buf.at[slot], out_hbm.at[wid, g], osem[slot])

    def wait_out(slot):
        pltpu.make_async_copy(
            buf.at[slot], out_hbm.at[wid, 0], osem[slot]
        ).wait()

    fire_gather(0, 0)
    fire_gather(1, 1)

    p_iters = n_groups // R

    def block(t, carry):
        for j in range(R):
            g = t * R + j
            s2 = (j + 2) % R
            if j < 2:
                @pl.when(t > 0)
                def _():
                    wait_out(s2)
                fire_gather(g + 2, s2)
            else:
                wait_out(s2)

                @pl.when(t < p_iters - 1)
                def _():
                    fire_gather(g + 2, s2)
            wait_gather(j)
            fire_out(g, j)
        return carry

    lax.fori_loop(0, p_iters, block, 0)
    wait_out(2)
    wait_out(3)


@jax.jit
def kernel(inputs, table):
    batch, hist = inputs.shape
    num_emb, dim = table.shape
    total = batch * hist
    assert total % (NW * GROUP) == 0
    n_groups = total // (NW * GROUP)
    assert n_groups % R == 0
    n_chunks = n_groups * K

    idx = inputs.reshape(NW, n_chunks, CHUNK).astype(jnp.int32)

    mesh = plsc.VectorSubcoreMesh(core_axis_name="c", subcore_axis_name="s")
    out = pl.kernel(
        functools.partial(_gather_body, n_groups, dim),
        out_type=jax.ShapeDtypeStruct((NW, n_groups, GROUP, dim), jnp.float32),
        mesh=mesh,
        scratch_types=[
            pltpu.VMEM((n_chunks, CHUNK), jnp.int32),
            pltpu.VMEM((R, GROUP, dim), jnp.float32),
        ] + [pltpu.SemaphoreType.DMA] * (2 * R),
        compiler_params=pltpu.CompilerParams(use_tc_tiling_on_sc=False),
    )(idx, table)

    return out.reshape(batch, hist, dim)
